# baseline (speedup 1.0000x reference)
import functools

import jax
import jax.numpy as jnp
from jax import lax
from jax.experimental import pallas as pl
from jax.experimental.pallas import tpu as pltpu
from jax.experimental.pallas import tpu_sc as plsc

N = 16384
D = 64
F = 4
NC = 2
NS = 16
NW = NC * NS
WPF = NW // F
DBLK = D // WPF
NCHUNK = 2048
NTASK = N // NCHUNK
GCH = 128
GPC = NCHUNK // GCH


def _sc_embed(field_a, field_b, field_c, field_d, wta, wtb, wtc, wtd):
    mesh = plsc.VectorSubcoreMesh(core_axis_name="c", subcore_axis_name="s")

    @functools.partial(
        pl.kernel,
        out_type=jax.ShapeDtypeStruct((F, D, N), jnp.float32),
        mesh=mesh,
        scratch_types=[
            pltpu.VMEM((NCHUNK,), jnp.int32),
            pltpu.VMEM((DBLK, NCHUNK), jnp.float32),
            pltpu.SemaphoreType.DMA,
        ],
        compiler_params=pltpu.CompilerParams(use_tc_tiling_on_sc=False),
    )
    def k(ia, ib, ic, idd, ta, tb, tc, td, out, ichunk, stage, gsem):
        wid = lax.axis_index("s") * NC + lax.axis_index("c")
        fld = wid // WPF
        a = wid % WPF
        idx_hbm = (ia, ib, ic, idd)
        tables = (ta, tb, tc, td)

        for fs in range(F):
            @pl.when(fld == fs)
            def _():
                wt = tables[fs]

                def task(nc, _):
                    pltpu.sync_copy(
                        idx_hbm[fs].at[pl.ds(nc * NCHUNK, NCHUNK)], ichunk)

                    def per_d(dr, _):
                        d = a * DBLK + dr
                        hs = []
                        for c in range(GPC):
                            hs.append(pltpu.async_copy(
                                wt.at[d].at[ichunk.at[pl.ds(c * GCH, GCH)]],
                                stage.at[dr, pl.ds(c * GCH, GCH)], gsem))
                        for h in hs:
                            h.wait()
                        return ()

                    lax.fori_loop(0, DBLK, per_d, (), unroll=False)
                    pltpu.sync_copy(
                        stage,
                        out.at[fs, pl.ds(a * DBLK, DBLK),
                               pl.ds(nc * NCHUNK, NCHUNK)])
                    return ()

                lax.fori_loop(0, NTASK, task, (), unroll=False)

    return k(field_a, field_b, field_c, field_d, wta, wtb, wtc, wtd)


def kernel(field_a, field_b, field_c, field_d,
           W_field_a, W_field_b, W_field_c, W_field_d):
    res = _sc_embed(field_a, field_b, field_c, field_d,
                    W_field_a.T, W_field_b.T, W_field_c.T, W_field_d.T)
    return jnp.transpose(res, (2, 0, 1))


# speedup vs baseline: 1.1152x; 1.1152x over previous
import functools

import jax
import jax.numpy as jnp
from jax import lax
from jax.experimental import pallas as pl
from jax.experimental.pallas import tpu as pltpu
from jax.experimental.pallas import tpu_sc as plsc

N = 16384
D = 64
NC = 2
NS = 16
NW = NC * NS
PPW = 2
ICH = 2048
VA = 1000000
VB = 100000
VD = 1000


def _stride(vocab):
    return (vocab + 1023) // 1024 * 1024


def _tc_linearize(wt, vocab):
    sv = _stride(vocab)
    nt = sv // 1024

    def body(in_ref, out_ref):
        out_ref[...] = in_ref[...].reshape(out_ref.shape)

    out4 = pl.pallas_call(
        body,
        grid=(D // 8, nt),
        in_specs=[pl.BlockSpec((8, 1024), lambda a, t: (a, t))],
        out_specs=pl.BlockSpec((8, 1, 8, 128), lambda a, t: (a, t, 0, 0)),
        out_shape=jax.ShapeDtypeStruct((D, nt, 8, 128), jnp.float32),
    )(wt)
    return out4.reshape(D * sv)


def _sc_bcd(field_b, field_c, field_d, linb, linc, lind):
    mesh = plsc.VectorSubcoreMesh(core_axis_name="c", subcore_axis_name="s")

    @functools.partial(
        pl.kernel,
        out_type=jax.ShapeDtypeStruct((3, D, N), jnp.float32),
        mesh=mesh,
        scratch_types=[
            pltpu.VMEM((VB,), jnp.float32),
            pltpu.VMEM((ICH,), jnp.int32),
            pltpu.VMEM((ICH,), jnp.float32),
        ],
        compiler_params=pltpu.CompilerParams(use_tc_tiling_on_sc=False,
                                             needs_layout_passes=False),
    )
    def k(ib, ic, idd, tb, tc, td, out, s_plane, s_ichunk, s_row):
        wid = lax.axis_index("s") * NC + lax.axis_index("c")

        def field_planes(fs, idx_hbm, lin, vocab):
            def plane(p, _):
                d = wid * PPW + p
                pltpu.sync_copy(lin.at[pl.ds(d * _stride(vocab), vocab)],
                                s_plane.at[pl.ds(0, vocab)])

                def chunk(ch, _):
                    pltpu.sync_copy(
                        idx_hbm.at[pl.ds(ch * ICH, ICH)], s_ichunk)

                    def group(g, _):
                        v16 = s_ichunk[pl.ds(g * 16, 16)]
                        s_row[pl.ds(g * 16, 16)] = plsc.load_gather(
                            s_plane, [v16])
                        return ()

                    lax.fori_loop(0, ICH // 16, group, (), unroll=4)
                    pltpu.sync_copy(
                        s_row, out.at[fs, d, pl.ds(ch * ICH, ICH)])
                    return ()

                lax.fori_loop(0, N // ICH, chunk, (), unroll=False)
                return ()

            lax.fori_loop(0, PPW, plane, (), unroll=False)

        field_planes(0, ib, tb, VB)
        field_planes(1, ic, tc, VB)
        field_planes(2, idd, td, VD)

    return k(field_b, field_c, field_d, linb, linc, lind)


def _sc_a(field_a, lin):
    mesh = plsc.VectorSubcoreMesh(core_axis_name="c", subcore_axis_name="s")

    @functools.partial(
        pl.kernel,
        out_type=jax.ShapeDtypeStruct((D, N), jnp.float32),
        mesh=mesh,
        scratch_types=[
            pltpu.VMEM((N,), jnp.int32),
            pltpu.VMEM((N,), jnp.float32),
            pltpu.SemaphoreType.DMA,
        ],
        compiler_params=pltpu.CompilerParams(use_tc_tiling_on_sc=False,
                                             needs_layout_passes=False),
    )
    def k(ia, tl, out, s_idx, s_row, gsem):
        wid = lax.axis_index("s") * NC + lax.axis_index("c")
        pltpu.sync_copy(ia, s_idx)

        def plane(p, _):
            d = wid * PPW + p
            row = tl.at[pl.ds(d * _stride(VA), VA)]

            def batch(b, _):
                hs = []
                for i in range(16):
                    ch = b * 16 + i
                    hs.append(pltpu.async_copy(
                        row.at[s_idx.at[pl.ds(ch * 128, 128)]],
                        s_row.at[pl.ds(ch * 128, 128)], gsem))
                for h in hs:
                    h.wait()
                return ()

            lax.fori_loop(0, N // 128 // 16, batch, (), unroll=False)
            pltpu.sync_copy(s_row, out.at[d])
            return ()

        lax.fori_loop(0, PPW, plane, (), unroll=False)

    return k(field_a, lin)


def kernel(field_a, field_b, field_c, field_d,
           W_field_a, W_field_b, W_field_c, W_field_d):
    linb = _tc_linearize(W_field_b.T, VB)
    linc = _tc_linearize(W_field_c.T, VB)
    lind = _tc_linearize(W_field_d.T, VD)
    lina = _tc_linearize(W_field_a.T, VA)
    out_bcd = _sc_bcd(field_b, field_c, field_d, linb, linc, lind)
    out_a = _sc_a(field_a, lina)
    res = jnp.concatenate([out_a[None], out_bcd], axis=0)
    return jnp.transpose(res, (2, 0, 1))


# speedup vs baseline: 12.4668x; 11.1793x over previous
import functools

import jax
import jax.numpy as jnp
from jax import lax
from jax.experimental import pallas as pl
from jax.experimental.pallas import tpu as pltpu
from jax.experimental.pallas import tpu_sc as plsc

N = 16384
D = 64
NC = 2
NS = 16
NW = NC * NS
PPW = 2
ICH = 2048
VA = 1000000
VB = 100000
VD = 1000


def _stride(vocab):
    return (vocab + 1023) // 1024 * 1024


def _tc_linearize(wt, vocab):
    sv = _stride(vocab)
    nt = sv // 1024
    kt = min(nt, 64)

    def body(in_ref, out_ref):
        out_ref[...] = in_ref[...].reshape(out_ref.shape)

    out4 = pl.pallas_call(
        body,
        grid=(D // 8, (nt + kt - 1) // kt),
        in_specs=[pl.BlockSpec((8, kt * 1024), lambda a, t: (a, t))],
        out_specs=pl.BlockSpec((8, kt, 8, 128), lambda a, t: (a, t, 0, 0)),
        out_shape=jax.ShapeDtypeStruct((D, nt, 8, 128), jnp.float32),
    )(wt)
    return out4.reshape(D * sv)


def _sc_bcd(field_b, field_c, field_d, linb, linc, lind):
    mesh = plsc.VectorSubcoreMesh(core_axis_name="c", subcore_axis_name="s")

    @functools.partial(
        pl.kernel,
        out_type=jax.ShapeDtypeStruct((3, D, N), jnp.float32),
        mesh=mesh,
        scratch_types=[
            pltpu.VMEM((VB,), jnp.float32),
            pltpu.VMEM((ICH,), jnp.int32),
            pltpu.VMEM((ICH,), jnp.float32),
        ],
        compiler_params=pltpu.CompilerParams(use_tc_tiling_on_sc=False,
                                             needs_layout_passes=False),
    )
    def k(ib, ic, idd, tb, tc, td, out, s_plane, s_ichunk, s_row):
        wid = lax.axis_index("s") * NC + lax.axis_index("c")

        def field_planes(fs, idx_hbm, lin, vocab):
            def plane(p, _):
                d = wid * PPW + p
                pltpu.sync_copy(lin.at[pl.ds(d * _stride(vocab), vocab)],
                                s_plane.at[pl.ds(0, vocab)])

                def chunk(ch, _):
                    pltpu.sync_copy(
                        idx_hbm.at[pl.ds(ch * ICH, ICH)], s_ichunk)

                    def group(g, _):
                        v16 = s_ichunk[pl.ds(g * 16, 16)]
                        s_row[pl.ds(g * 16, 16)] = plsc.load_gather(
                            s_plane, [v16])
                        return ()

                    lax.fori_loop(0, ICH // 16, group, (), unroll=4)
                    pltpu.sync_copy(
                        s_row, out.at[fs, d, pl.ds(ch * ICH, ICH)])
                    return ()

                lax.fori_loop(0, N // ICH, chunk, (), unroll=False)
                return ()

            lax.fori_loop(0, PPW, plane, (), unroll=False)

        field_planes(0, ib, tb, VB)
        field_planes(1, ic, tc, VB)
        field_planes(2, idd, td, VD)

    return k(field_b, field_c, field_d, linb, linc, lind)


def _sc_a(field_a, lin):
    mesh = plsc.VectorSubcoreMesh(core_axis_name="c", subcore_axis_name="s")

    @functools.partial(
        pl.kernel,
        out_type=jax.ShapeDtypeStruct((D, N), jnp.float32),
        mesh=mesh,
        scratch_types=[
            pltpu.VMEM((N,), jnp.int32),
            pltpu.VMEM((N,), jnp.float32),
            pltpu.SemaphoreType.DMA,
        ],
        compiler_params=pltpu.CompilerParams(use_tc_tiling_on_sc=False,
                                             needs_layout_passes=False),
    )
    def k(ia, tl, out, s_idx, s_row, gsem):
        wid = lax.axis_index("s") * NC + lax.axis_index("c")
        pltpu.sync_copy(ia, s_idx)

        def plane(p, _):
            d = wid * PPW + p
            row = tl.at[pl.ds(d * _stride(VA), VA)]

            def batch(b, _):
                hs = []
                for i in range(16):
                    ch = b * 16 + i
                    hs.append(pltpu.async_copy(
                        row.at[s_idx.at[pl.ds(ch * 128, 128)]],
                        s_row.at[pl.ds(ch * 128, 128)], gsem))
                for h in hs:
                    h.wait()
                return ()

            lax.fori_loop(0, N // 128 // 16, batch, (), unroll=False)
            pltpu.sync_copy(s_row, out.at[d])
            return ()

        lax.fori_loop(0, PPW, plane, (), unroll=False)

    return k(field_a, lin)


def kernel(field_a, field_b, field_c, field_d,
           W_field_a, W_field_b, W_field_c, W_field_d):
    linb = _tc_linearize(W_field_b.T, VB)
    linc = _tc_linearize(W_field_c.T, VB)
    lind = _tc_linearize(W_field_d.T, VD)
    lina = _tc_linearize(W_field_a.T, VA)
    out_bcd = _sc_bcd(field_b, field_c, field_d, linb, linc, lind)
    out_a = _sc_a(field_a, lina)
    res = jnp.concatenate([out_a[None], out_bcd], axis=0)
    return jnp.transpose(res, (2, 0, 1))


# speedup vs baseline: 13.0224x; 1.0446x over previous
import functools

import jax
import jax.numpy as jnp
from jax import lax
from jax.experimental import pallas as pl
from jax.experimental.pallas import tpu as pltpu
from jax.experimental.pallas import tpu_sc as plsc

N = 16384
D = 64
NC = 2
NS = 16
NW = NC * NS
PPW = 2
ICH = 4096
VA = 1000000
VB = 100000
VD = 1000


def _stride(vocab):
    return (vocab + 1023) // 1024 * 1024


def _tc_linearize(wt, vocab):
    sv = _stride(vocab)
    nt = sv // 1024
    kt = min(nt, 64)

    def body(in_ref, out_ref):
        out_ref[...] = in_ref[...].reshape(out_ref.shape)

    out4 = pl.pallas_call(
        body,
        grid=(D // 8, (nt + kt - 1) // kt),
        in_specs=[pl.BlockSpec((8, kt * 1024), lambda a, t: (a, t))],
        out_specs=pl.BlockSpec((8, kt, 8, 128), lambda a, t: (a, t, 0, 0)),
        out_shape=jax.ShapeDtypeStruct((D, nt, 8, 128), jnp.float32),
    )(wt)
    return out4.reshape(D * sv)


def _sc_bcd(field_b, field_c, field_d, linb, linc, lind):
    mesh = plsc.VectorSubcoreMesh(core_axis_name="c", subcore_axis_name="s")

    @functools.partial(
        pl.kernel,
        out_type=jax.ShapeDtypeStruct((3, D, N), jnp.float32),
        mesh=mesh,
        scratch_types=[
            pltpu.VMEM((VB,), jnp.float32),
            pltpu.VMEM((ICH,), jnp.int32),
            pltpu.VMEM((ICH,), jnp.float32),
        ],
        compiler_params=pltpu.CompilerParams(use_tc_tiling_on_sc=False,
                                             needs_layout_passes=False),
    )
    def k(ib, ic, idd, tb, tc, td, out, s_plane, s_ichunk, s_row):
        wid = lax.axis_index("s") * NC + lax.axis_index("c")

        def field_planes(fs, idx_hbm, lin, vocab):
            def plane(p, _):
                d = wid * PPW + p
                pltpu.sync_copy(lin.at[pl.ds(d * _stride(vocab), vocab)],
                                s_plane.at[pl.ds(0, vocab)])

                def chunk(ch, _):
                    pltpu.sync_copy(
                        idx_hbm.at[pl.ds(ch * ICH, ICH)], s_ichunk)

                    def group(g, _):
                        v16 = s_ichunk[pl.ds(g * 16, 16)]
                        s_row[pl.ds(g * 16, 16)] = plsc.load_gather(
                            s_plane, [v16])
                        return ()

                    lax.fori_loop(0, ICH // 16, group, (), unroll=16)
                    pltpu.sync_copy(
                        s_row, out.at[fs, d, pl.ds(ch * ICH, ICH)])
                    return ()

                lax.fori_loop(0, N // ICH, chunk, (), unroll=False)
                return ()

            lax.fori_loop(0, PPW, plane, (), unroll=False)

        field_planes(0, ib, tb, VB)
        field_planes(1, ic, tc, VB)
        field_planes(2, idd, td, VD)

    return k(field_b, field_c, field_d, linb, linc, lind)


def _sc_a(field_a, lin):
    mesh = plsc.VectorSubcoreMesh(core_axis_name="c", subcore_axis_name="s")

    @functools.partial(
        pl.kernel,
        out_type=jax.ShapeDtypeStruct((D, N), jnp.float32),
        mesh=mesh,
        scratch_types=[
            pltpu.VMEM((N,), jnp.int32),
            pltpu.VMEM((N,), jnp.float32),
            pltpu.SemaphoreType.DMA,
        ],
        compiler_params=pltpu.CompilerParams(use_tc_tiling_on_sc=False,
                                             needs_layout_passes=False),
    )
    def k(ia, tl, out, s_idx, s_row, gsem):
        wid = lax.axis_index("s") * NC + lax.axis_index("c")
        pltpu.sync_copy(ia, s_idx)

        def plane(p, _):
            d = wid * PPW + p
            row = tl.at[pl.ds(d * _stride(VA), VA)]

            def batch(b, _):
                hs = []
                for i in range(32):
                    ch = b * 32 + i
                    hs.append(pltpu.async_copy(
                        row.at[s_idx.at[pl.ds(ch * 128, 128)]],
                        s_row.at[pl.ds(ch * 128, 128)], gsem))
                for h in hs:
                    h.wait()
                return ()

            lax.fori_loop(0, N // 128 // 32, batch, (), unroll=False)
            pltpu.sync_copy(s_row, out.at[d])
            return ()

        lax.fori_loop(0, PPW, plane, (), unroll=False)

    return k(field_a, lin)


def kernel(field_a, field_b, field_c, field_d,
           W_field_a, W_field_b, W_field_c, W_field_d):
    linb = _tc_linearize(W_field_b.T, VB)
    linc = _tc_linearize(W_field_c.T, VB)
    lind = _tc_linearize(W_field_d.T, VD)
    lina = _tc_linearize(W_field_a.T, VA)
    out_bcd = _sc_bcd(field_b, field_c, field_d, linb, linc, lind)
    out_a = _sc_a(field_a, lina)
    res = jnp.concatenate([out_a[None], out_bcd], axis=0)
    return jnp.transpose(res, (2, 0, 1))


# speedup vs baseline: 16.0680x; 1.2339x over previous
import functools

import jax
import jax.numpy as jnp
from jax import lax
from jax.experimental import pallas as pl
from jax.experimental.pallas import tpu as pltpu
from jax.experimental.pallas import tpu_sc as plsc

N = 16384
D = 64
NC = 2
NS = 16
NW = NC * NS
PPW = 2
ICH = 4096
VA = 1000000
VB = 100000
VD = 1000


def _stride(vocab):
    return (vocab + 1023) // 1024 * 1024


def _tc_linearize(wt, vocab):
    sv = _stride(vocab)
    nt = sv // 1024
    kt = min(nt, 128)

    def body(in_ref, out_ref):
        out_ref[...] = in_ref[...].reshape(out_ref.shape)

    out4 = pl.pallas_call(
        body,
        grid=(D // 8, (nt + kt - 1) // kt),
        in_specs=[pl.BlockSpec((8, kt * 1024), lambda a, t: (a, t))],
        out_specs=pl.BlockSpec((8, kt, 8, 128), lambda a, t: (a, t, 0, 0)),
        out_shape=jax.ShapeDtypeStruct((D, nt, 8, 128), jnp.float32),
    )(wt)
    return out4.reshape(D * sv)


def _sc_bcd(field_b, field_c, field_d, linb, linc, lind):
    mesh = plsc.VectorSubcoreMesh(core_axis_name="c", subcore_axis_name="s")

    @functools.partial(
        pl.kernel,
        out_type=jax.ShapeDtypeStruct((3, D, N), jnp.float32),
        mesh=mesh,
        scratch_types=[
            pltpu.VMEM((VB,), jnp.float32),
            pltpu.VMEM((ICH,), jnp.int32),
            pltpu.VMEM((ICH,), jnp.float32),
        ],
        compiler_params=pltpu.CompilerParams(use_tc_tiling_on_sc=False,
                                             needs_layout_passes=False),
    )
    def k(ib, ic, idd, tb, tc, td, out, s_plane, s_ichunk, s_row):
        wid = lax.axis_index("s") * NC + lax.axis_index("c")

        def field_planes(fs, idx_hbm, lin, vocab):
            def plane(p, _):
                d = wid * PPW + p
                pltpu.sync_copy(lin.at[pl.ds(d * _stride(vocab), vocab)],
                                s_plane.at[pl.ds(0, vocab)])

                def chunk(ch, _):
                    pltpu.sync_copy(
                        idx_hbm.at[pl.ds(ch * ICH, ICH)], s_ichunk)

                    def group(g, _):
                        v16 = s_ichunk[pl.ds(g * 16, 16)]
                        s_row[pl.ds(g * 16, 16)] = plsc.load_gather(
                            s_plane, [v16])
                        return ()

                    lax.fori_loop(0, ICH // 16, group, (), unroll=16)
                    pltpu.sync_copy(
                        s_row, out.at[fs, d, pl.ds(ch * ICH, ICH)])
                    return ()

                lax.fori_loop(0, N // ICH, chunk, (), unroll=False)
                return ()

            lax.fori_loop(0, PPW, plane, (), unroll=False)

        field_planes(0, ib, tb, VB)
        field_planes(1, ic, tc, VB)
        field_planes(2, idd, td, VD)

    return k(field_b, field_c, field_d, linb, linc, lind)


def _sc_a(field_a, lin):
    mesh = plsc.VectorSubcoreMesh(core_axis_name="c", subcore_axis_name="s")

    @functools.partial(
        pl.kernel,
        out_type=jax.ShapeDtypeStruct((D, N), jnp.float32),
        mesh=mesh,
        scratch_types=[
            pltpu.VMEM((N,), jnp.int32),
            pltpu.VMEM((N,), jnp.float32),
            pltpu.SemaphoreType.DMA,
        ],
        compiler_params=pltpu.CompilerParams(use_tc_tiling_on_sc=False,
                                             needs_layout_passes=False),
    )
    def k(ia, tl, out, s_idx, s_row, gsem):
        wid = lax.axis_index("s") * NC + lax.axis_index("c")
        pltpu.sync_copy(ia, s_idx)

        def plane(p, _):
            d = wid * PPW + p
            row = tl.at[pl.ds(d * _stride(VA), VA)]

            def batch(b, _):
                hs = []
                for i in range(32):
                    ch = b * 32 + i
                    hs.append(pltpu.async_copy(
                        row.at[s_idx.at[pl.ds(ch * 128, 128)]],
                        s_row.at[pl.ds(ch * 128, 128)], gsem))
                for h in hs:
                    h.wait()
                return ()

            lax.fori_loop(0, N // 128 // 32, batch, (), unroll=False)
            pltpu.sync_copy(s_row, out.at[d])
            return ()

        lax.fori_loop(0, PPW, plane, (), unroll=False)

    return k(field_a, lin)


def kernel(field_a, field_b, field_c, field_d,
           W_field_a, W_field_b, W_field_c, W_field_d):
    linb = _tc_linearize(W_field_b.T, VB)
    linc = _tc_linearize(W_field_c.T, VB)
    lind = _tc_linearize(W_field_d.T, VD)
    wta, linb, linc, lind = lax.optimization_barrier(
        (W_field_a.T, linb, linc, lind))
    lina = _tc_linearize(wta, VA)
    out_bcd = _sc_bcd(field_b, field_c, field_d, linb, linc, lind)
    out_a = _sc_a(field_a, lina)
    res = jnp.concatenate([out_a[None], out_bcd], axis=0)
    return jnp.transpose(res, (2, 0, 1))


# speedup vs baseline: 16.2193x; 1.0094x over previous
import functools

import jax
import jax.numpy as jnp
from jax import lax
from jax.experimental import pallas as pl
from jax.experimental.pallas import tpu as pltpu
from jax.experimental.pallas import tpu_sc as plsc

N = 16384
D = 64
NC = 2
NS = 16
NW = NC * NS
PPW = 2
ICH = 4096
VA = 1000000
VB = 100000
VD = 1000


def _stride(vocab):
    return (vocab + 1023) // 1024 * 1024


def _tc_linearize(wt, vocab, d0=0, nd=D):
    sv = _stride(vocab)
    nt = sv // 1024
    kt = min(nt, 128)
    a0 = d0 // 8

    def body(in_ref, out_ref):
        out_ref[...] = in_ref[...].reshape(out_ref.shape)

    out4 = pl.pallas_call(
        body,
        grid=(nd // 8, (nt + kt - 1) // kt),
        in_specs=[pl.BlockSpec((8, kt * 1024), lambda a, t: (a + a0, t))],
        out_specs=pl.BlockSpec((8, kt, 8, 128), lambda a, t: (a, t, 0, 0)),
        out_shape=jax.ShapeDtypeStruct((nd, nt, 8, 128), jnp.float32),
    )(wt)
    return out4.reshape(nd * sv)


def _sc_bcd(field_b, field_c, field_d, linb, linc, lind):
    mesh = plsc.VectorSubcoreMesh(core_axis_name="c", subcore_axis_name="s")

    @functools.partial(
        pl.kernel,
        out_type=jax.ShapeDtypeStruct((3, D, N), jnp.float32),
        mesh=mesh,
        scratch_types=[
            pltpu.VMEM((VB,), jnp.float32),
            pltpu.VMEM((ICH,), jnp.int32),
            pltpu.VMEM((ICH,), jnp.float32),
        ],
        compiler_params=pltpu.CompilerParams(use_tc_tiling_on_sc=False,
                                             needs_layout_passes=False),
    )
    def k(ib, ic, idd, tb, tc, td, out, s_plane, s_ichunk, s_row):
        wid = lax.axis_index("s") * NC + lax.axis_index("c")

        def field_planes(fs, idx_hbm, lin, vocab):
            def plane(p, _):
                d = wid * PPW + p
                pltpu.sync_copy(lin.at[pl.ds(d * _stride(vocab), vocab)],
                                s_plane.at[pl.ds(0, vocab)])

                def chunk(ch, _):
                    pltpu.sync_copy(
                        idx_hbm.at[pl.ds(ch * ICH, ICH)], s_ichunk)

                    def group(g, _):
                        v16 = s_ichunk[pl.ds(g * 16, 16)]
                        s_row[pl.ds(g * 16, 16)] = plsc.load_gather(
                            s_plane, [v16])
                        return ()

                    lax.fori_loop(0, ICH // 16, group, (), unroll=16)
                    pltpu.sync_copy(
                        s_row, out.at[fs, d, pl.ds(ch * ICH, ICH)])
                    return ()

                lax.fori_loop(0, N // ICH, chunk, (), unroll=False)
                return ()

            lax.fori_loop(0, PPW, plane, (), unroll=False)

        field_planes(0, ib, tb, VB)
        field_planes(1, ic, tc, VB)
        field_planes(2, idd, td, VD)

    return k(field_b, field_c, field_d, linb, linc, lind)


def _sc_a(field_a, lin):
    mesh = plsc.VectorSubcoreMesh(core_axis_name="c", subcore_axis_name="s")

    @functools.partial(
        pl.kernel,
        out_type=jax.ShapeDtypeStruct((D // 2, N), jnp.float32),
        mesh=mesh,
        scratch_types=[
            pltpu.VMEM((N,), jnp.int32),
            pltpu.VMEM((N,), jnp.float32),
            pltpu.SemaphoreType.DMA,
        ],
        compiler_params=pltpu.CompilerParams(use_tc_tiling_on_sc=False,
                                             needs_layout_passes=False),
    )
    def k(ia, tl, out, s_idx, s_row, gsem):
        wid = lax.axis_index("s") * NC + lax.axis_index("c")
        pltpu.sync_copy(ia, s_idx)
        d = wid
        row = tl.at[pl.ds(d * _stride(VA), VA)]

        def batch(b, _):
            hs = []
            for i in range(32):
                ch = b * 32 + i
                hs.append(pltpu.async_copy(
                    row.at[s_idx.at[pl.ds(ch * 128, 128)]],
                    s_row.at[pl.ds(ch * 128, 128)], gsem))
            for h in hs:
                h.wait()
            return ()

        lax.fori_loop(0, N // 128 // 32, batch, (), unroll=False)
        pltpu.sync_copy(s_row, out.at[d])

    return k(field_a, lin)


def kernel(field_a, field_b, field_c, field_d,
           W_field_a, W_field_b, W_field_c, W_field_d):
    linb = _tc_linearize(W_field_b.T, VB)
    linc = _tc_linearize(W_field_c.T, VB)
    lind = _tc_linearize(W_field_d.T, VD)
    wta, linb, linc, lind = lax.optimization_barrier(
        (W_field_a.T, linb, linc, lind))
    lina1 = _tc_linearize(wta, VA, 0, D // 2)
    wta2, lina1 = lax.optimization_barrier((wta, lina1))
    lina2 = _tc_linearize(wta2, VA, D // 2, D // 2)
    out_bcd = _sc_bcd(field_b, field_c, field_d, linb, linc, lind)
    out_a1 = _sc_a(field_a, lina1)
    out_a2 = _sc_a(field_a, lina2)
    res = jnp.concatenate(
        [out_a1.reshape(1, D // 2, N), out_a2.reshape(1, D // 2, N)], axis=1)
    res = jnp.concatenate([res, out_bcd], axis=0)
    return jnp.transpose(res, (2, 0, 1))


# speedup vs baseline: 16.3117x; 1.0057x over previous
import functools

import jax
import jax.numpy as jnp
from jax import lax
from jax.experimental import pallas as pl
from jax.experimental.pallas import tpu as pltpu
from jax.experimental.pallas import tpu_sc as plsc

N = 16384
D = 64
NC = 2
NS = 16
NW = NC * NS
PPW = 2
ICH = 4096
VA = 1000000
VB = 100000
VD = 1000


def _stride(vocab):
    return (vocab + 1023) // 1024 * 1024


def _tc_linearize(wt, vocab, d0=0, nd=D):
    sv = _stride(vocab)
    nt = sv // 1024
    kt = min(nt, 128)
    a0 = d0 // 8

    def body(in_ref, out_ref):
        out_ref[...] = in_ref[...].reshape(out_ref.shape)

    out4 = pl.pallas_call(
        body,
        grid=(nd // 8, (nt + kt - 1) // kt),
        in_specs=[pl.BlockSpec((8, kt * 1024), lambda a, t: (a + a0, t))],
        out_specs=pl.BlockSpec((8, kt, 8, 128), lambda a, t: (a, t, 0, 0)),
        out_shape=jax.ShapeDtypeStruct((nd, nt, 8, 128), jnp.float32),
    )(wt)
    return out4.reshape(nd * sv)


def _sc_bcd(field_b, field_c, field_d, linb, linc, lind):
    mesh = plsc.VectorSubcoreMesh(core_axis_name="c", subcore_axis_name="s")

    @functools.partial(
        pl.kernel,
        out_type=jax.ShapeDtypeStruct((3, D, N), jnp.float32),
        mesh=mesh,
        scratch_types=[
            pltpu.VMEM((VB,), jnp.float32),
            pltpu.VMEM((ICH,), jnp.int32),
            pltpu.VMEM((ICH,), jnp.float32),
        ],
        compiler_params=pltpu.CompilerParams(use_tc_tiling_on_sc=False,
                                             needs_layout_passes=False),
    )
    def k(ib, ic, idd, tb, tc, td, out, s_plane, s_ichunk, s_row):
        wid = lax.axis_index("s") * NC + lax.axis_index("c")

        def field_planes(fs, idx_hbm, lin, vocab):
            def plane(p, _):
                d = wid * PPW + p
                pltpu.sync_copy(lin.at[pl.ds(d * _stride(vocab), vocab)],
                                s_plane.at[pl.ds(0, vocab)])

                def chunk(ch, _):
                    pltpu.sync_copy(
                        idx_hbm.at[pl.ds(ch * ICH, ICH)], s_ichunk)

                    def group(g, _):
                        v16 = s_ichunk[pl.ds(g * 16, 16)]
                        s_row[pl.ds(g * 16, 16)] = plsc.load_gather(
                            s_plane, [v16])
                        return ()

                    lax.fori_loop(0, ICH // 16, group, (), unroll=8)
                    pltpu.sync_copy(
                        s_row, out.at[fs, d, pl.ds(ch * ICH, ICH)])
                    return ()

                lax.fori_loop(0, N // ICH, chunk, (), unroll=False)
                return ()

            lax.fori_loop(0, PPW, plane, (), unroll=False)

        field_planes(0, ib, tb, VB)
        field_planes(1, ic, tc, VB)
        field_planes(2, idd, td, VD)

    return k(field_b, field_c, field_d, linb, linc, lind)


def _sc_a(field_a, lin):
    mesh = plsc.VectorSubcoreMesh(core_axis_name="c", subcore_axis_name="s")

    @functools.partial(
        pl.kernel,
        out_type=jax.ShapeDtypeStruct((D // 2, N), jnp.float32),
        mesh=mesh,
        scratch_types=[
            pltpu.VMEM((N,), jnp.int32),
            pltpu.VMEM((N,), jnp.float32),
            pltpu.SemaphoreType.DMA,
        ],
        compiler_params=pltpu.CompilerParams(use_tc_tiling_on_sc=False,
                                             needs_layout_passes=False),
    )
    def k(ia, tl, out, s_idx, s_row, gsem):
        wid = lax.axis_index("s") * NC + lax.axis_index("c")
        pltpu.sync_copy(ia, s_idx)
        d = wid
        row = tl.at[pl.ds(d * _stride(VA), VA)]

        def batch(b, _):
            hs = []
            for i in range(32):
                ch = b * 32 + i
                hs.append(pltpu.async_copy(
                    row.at[s_idx.at[pl.ds(ch * 128, 128)]],
                    s_row.at[pl.ds(ch * 128, 128)], gsem))
            for h in hs:
                h.wait()
            return ()

        lax.fori_loop(0, N // 128 // 32, batch, (), unroll=False)
        pltpu.sync_copy(s_row, out.at[d])

    return k(field_a, lin)


def kernel(field_a, field_b, field_c, field_d,
           W_field_a, W_field_b, W_field_c, W_field_d):
    linb = _tc_linearize(W_field_b.T, VB)
    linc = _tc_linearize(W_field_c.T, VB)
    lind = _tc_linearize(W_field_d.T, VD)
    wta, linb, linc, lind = lax.optimization_barrier(
        (W_field_a.T, linb, linc, lind))
    lina1 = _tc_linearize(wta, VA, 0, D // 2)
    wta2, lina1 = lax.optimization_barrier((wta, lina1))
    lina2 = _tc_linearize(wta2, VA, D // 2, D // 2)
    out_bcd = _sc_bcd(field_b, field_c, field_d, linb, linc, lind)
    out_a1 = _sc_a(field_a, lina1)
    out_a2 = _sc_a(field_a, lina2)
    res = jnp.concatenate(
        [out_a1.reshape(1, D // 2, N), out_a2.reshape(1, D // 2, N)], axis=1)
    res = jnp.concatenate([res, out_bcd], axis=0)
    return jnp.transpose(res, (2, 0, 1))
